# initial kernel scaffold (unmeasured)
import jax
import jax.numpy as jnp
from jax import lax
from jax.experimental import pallas as pl
from jax.experimental.pallas import tpu as pltpu


def kernel(
    x,
):
    def body(*refs):
        pass

    out_shape = jax.ShapeDtypeStruct(..., jnp.float32)
    return pl.pallas_call(body, out_shape=out_shape)(...)


# baseline (device time: 84354 ns/iter reference)
import jax
import jax.numpy as jnp
from jax import lax
from jax.experimental import pallas as pl
from jax.experimental.pallas import tpu as pltpu

N_DEV = 4


def kernel(x):
    _, m, n_tot = x.shape
    n_out = n_tot // N_DEV

    def body(x_ref, out_ref, comm_ref, send_sems, recv_sems):
        my = lax.axis_index("i")
        left = (my - 1) % N_DEV
        right = (my + 1) % N_DEV

        barrier_sem = pltpu.get_barrier_semaphore()
        for nbr in (left, right):
            pl.semaphore_signal(
                barrier_sem, inc=1,
                device_id=(nbr,), device_id_type=pl.DeviceIdType.MESH,
            )
        pl.semaphore_wait(barrier_sem, 2)

        def chunk_bf16(c):
            return x_ref[0, :, pl.ds(c * n_out, n_out)].astype(jnp.bfloat16)

        comm_ref[3, :, :] = chunk_bf16(left)

        for h in range(N_DEV - 1):
            send_slot = 3 if h == 0 else h - 1
            rdma = pltpu.make_async_remote_copy(
                src_ref=comm_ref.at[send_slot],
                dst_ref=comm_ref.at[h],
                send_sem=send_sems.at[h],
                recv_sem=recv_sems.at[h],
                device_id=(right,),
                device_id_type=pl.DeviceIdType.MESH,
            )
            rdma.start()
            rdma.wait()

            c = (my - 2 - h) % N_DEV
            if h < N_DEV - 2:
                comm_ref[h, :, :] = comm_ref[h, :, :] + chunk_bf16(c)
            else:
                out_ref[:, :] = (
                    comm_ref[h, :, :].astype(jnp.float32)
                    + x_ref[0, :, pl.ds(my * n_out, n_out)]
                )

    return pl.pallas_call(
        body,
        out_shape=jax.ShapeDtypeStruct((m, n_out), jnp.float32),
        in_specs=[pl.BlockSpec(memory_space=pltpu.VMEM)],
        out_specs=pl.BlockSpec(memory_space=pltpu.VMEM),
        scratch_shapes=[
            pltpu.VMEM((4, m, n_out), jnp.bfloat16),
            pltpu.SemaphoreType.DMA((3,)),
            pltpu.SemaphoreType.DMA((3,)),
        ],
        compiler_params=pltpu.CompilerParams(collective_id=0),
    )(x)


# device time: 51196 ns/iter; 1.6477x vs baseline; 1.6477x over previous
import jax
import jax.numpy as jnp
from jax import lax
from jax.experimental import pallas as pl
from jax.experimental.pallas import tpu as pltpu

N_DEV = 4


def kernel(x):
    _, m, n_tot = x.shape
    n_out = n_tot // N_DEV
    half = n_out // 2

    def body(x_ref, out_ref, comm_r, comm_l, send_r, recv_r, send_l, recv_l):
        my = lax.axis_index("i")
        left = (my - 1) % N_DEV
        right = (my + 1) % N_DEV

        barrier_sem = pltpu.get_barrier_semaphore()
        for nbr in (left, right):
            pl.semaphore_signal(
                barrier_sem, inc=1,
                device_id=(nbr,), device_id_type=pl.DeviceIdType.MESH,
            )
        pl.semaphore_wait(barrier_sem, 2)

        def half0(c):
            return x_ref[0, :, pl.ds(c * n_out, half)].astype(jnp.bfloat16)

        def half1(c):
            return x_ref[0, :, pl.ds(c * n_out + half, half)].astype(jnp.bfloat16)

        comm_r[3, :, :] = half0(left)
        comm_l[3, :, :] = half1(right)

        for h in range(N_DEV - 1):
            ss = 3 if h == 0 else h - 1
            rdma_r = pltpu.make_async_remote_copy(
                src_ref=comm_r.at[ss],
                dst_ref=comm_r.at[h],
                send_sem=send_r.at[h],
                recv_sem=recv_r.at[h],
                device_id=(right,),
                device_id_type=pl.DeviceIdType.MESH,
            )
            rdma_l = pltpu.make_async_remote_copy(
                src_ref=comm_l.at[ss],
                dst_ref=comm_l.at[h],
                send_sem=send_l.at[h],
                recv_sem=recv_l.at[h],
                device_id=(left,),
                device_id_type=pl.DeviceIdType.MESH,
            )
            rdma_r.start()
            rdma_l.start()
            rdma_r.wait()
            rdma_l.wait()

            cr = (my - 2 - h) % N_DEV
            cl = (my + 2 + h) % N_DEV
            if h < N_DEV - 2:
                comm_r[h, :, :] = comm_r[h, :, :] + half0(cr)
                comm_l[h, :, :] = comm_l[h, :, :] + half1(cl)
            else:
                out_ref[:, :half] = (
                    comm_r[h, :, :].astype(jnp.float32)
                    + x_ref[0, :, pl.ds(my * n_out, half)]
                )
                out_ref[:, half:] = (
                    comm_l[h, :, :].astype(jnp.float32)
                    + x_ref[0, :, pl.ds(my * n_out + half, half)]
                )

    return pl.pallas_call(
        body,
        out_shape=jax.ShapeDtypeStruct((m, n_out), jnp.float32),
        in_specs=[pl.BlockSpec(memory_space=pltpu.VMEM)],
        out_specs=pl.BlockSpec(memory_space=pltpu.VMEM),
        scratch_shapes=[
            pltpu.VMEM((4, m, half), jnp.bfloat16),
            pltpu.VMEM((4, m, half), jnp.bfloat16),
            pltpu.SemaphoreType.DMA((3,)),
            pltpu.SemaphoreType.DMA((3,)),
            pltpu.SemaphoreType.DMA((3,)),
            pltpu.SemaphoreType.DMA((3,)),
        ],
        compiler_params=pltpu.CompilerParams(collective_id=0),
    )(x)


# device time: 45958 ns/iter; 1.8355x vs baseline; 1.1140x over previous
import jax
import jax.numpy as jnp
from jax import lax
from jax.experimental import pallas as pl
from jax.experimental.pallas import tpu as pltpu

N_DEV = 4
SEG = 4


def kernel(x):
    _, m, n_tot = x.shape
    n_out = n_tot // N_DEV
    half = n_out // 2
    mseg = m // SEG

    def body(x_ref, out_ref, comm_r, comm_l, contrib_r, contrib_l,
             send_r, recv_r, send_l, recv_l):
        my = lax.axis_index("i")
        left = (my - 1) % N_DEV
        right = (my + 1) % N_DEV

        barrier_sem = pltpu.get_barrier_semaphore()
        for nbr in (left, right):
            pl.semaphore_signal(
                barrier_sem, inc=1,
                device_id=(nbr,), device_id_type=pl.DeviceIdType.MESH,
            )
        pl.semaphore_wait(barrier_sem, 2)

        def half0(c, rows):
            return x_ref[0, rows, pl.ds(c * n_out, half)].astype(jnp.bfloat16)

        def half1(c, rows):
            return x_ref[0, rows, pl.ds(c * n_out + half, half)].astype(
                jnp.bfloat16)

        def rdma(direction, h, s):
            comm, ssem, rsem, dst_dev = (
                (comm_r, send_r, recv_r, right) if direction == 0
                else (comm_l, send_l, recv_l, left)
            )
            src_slot = 3 if h == 0 else h - 1
            rows = pl.ds(s * mseg, mseg)
            return pltpu.make_async_remote_copy(
                src_ref=comm.at[src_slot, rows, :],
                dst_ref=comm.at[h, rows, :],
                send_sem=ssem.at[h, s],
                recv_sem=rsem.at[h, s],
                device_id=(dst_dev,),
                device_id_type=pl.DeviceIdType.MESH,
            )

        full = pl.ds(0, m)

        comm_r[3, :, :] = half0(left, full)
        comm_l[3, :, :] = half1(right, full)
        for s in range(SEG):
            rdma(0, 0, s).start()
            rdma(1, 0, s).start()

        contrib_r[0, :, :] = half0((my - 2) % N_DEV, full)
        contrib_l[0, :, :] = half1((my + 2) % N_DEV, full)
        contrib_r[1, :, :] = half0((my - 3) % N_DEV, full)
        contrib_l[1, :, :] = half1((my + 3) % N_DEV, full)

        for h in range(N_DEV - 2):
            for s in range(SEG):
                rows = pl.ds(s * mseg, mseg)
                rdma(0, h, s).wait_recv()
                comm_r[h, rows, :] = (
                    comm_r[h, rows, :] + contrib_r[h, rows, :]
                )
                rdma(0, h + 1, s).start()
                rdma(1, h, s).wait_recv()
                comm_l[h, rows, :] = (
                    comm_l[h, rows, :] + contrib_l[h, rows, :]
                )
                rdma(1, h + 1, s).start()

        hl = N_DEV - 2
        for s in range(SEG):
            rows = pl.ds(s * mseg, mseg)
            rdma(0, hl, s).wait_recv()
            out_ref[rows, :half] = (
                comm_r[hl, rows, :].astype(jnp.float32)
                + x_ref[0, rows, pl.ds(my * n_out, half)]
            )
            rdma(1, hl, s).wait_recv()
            out_ref[rows, half:] = (
                comm_l[hl, rows, :].astype(jnp.float32)
                + x_ref[0, rows, pl.ds(my * n_out + half, half)]
            )

        for h in range(N_DEV - 1):
            for s in range(SEG):
                rdma(0, h, s).wait_send()
                rdma(1, h, s).wait_send()

    return pl.pallas_call(
        body,
        out_shape=jax.ShapeDtypeStruct((m, n_out), jnp.float32),
        in_specs=[pl.BlockSpec(memory_space=pltpu.VMEM)],
        out_specs=pl.BlockSpec(memory_space=pltpu.VMEM),
        scratch_shapes=[
            pltpu.VMEM((4, m, half), jnp.bfloat16),
            pltpu.VMEM((4, m, half), jnp.bfloat16),
            pltpu.VMEM((2, m, half), jnp.bfloat16),
            pltpu.VMEM((2, m, half), jnp.bfloat16),
            pltpu.SemaphoreType.DMA((3, SEG)),
            pltpu.SemaphoreType.DMA((3, SEG)),
            pltpu.SemaphoreType.DMA((3, SEG)),
            pltpu.SemaphoreType.DMA((3, SEG)),
        ],
        compiler_params=pltpu.CompilerParams(collective_id=0),
    )(x)
